# baseline (speedup 1.0000x reference)
import functools

import jax
import jax.numpy as jnp
from jax import lax
from jax.experimental import pallas as pl
from jax.experimental.pallas import tpu as pltpu
from jax.experimental.pallas import tpu_sc as plsc

N0, N1, N2 = 10000, 5000, 2500
D, HID, OUT = 128, 256, 64
K1 = 5120
M1 = 2560
K2 = 2560
M2 = 2560
BR = 256

DUMP_N = 16384
E1P, E2P = 327680, 163840
SLAB1, R1_, CH1, ZB1 = 256 * K1, 5, 5120, 10240
SLAB2, R2_, CH2, ZB2 = 640 * K2, 2, 2560, 5120


def _ln(h, g, b):
    mu = jnp.mean(h, axis=-1, keepdims=True)
    var = jnp.mean((h - mu) * (h - mu), axis=-1, keepdims=True)
    return (h - mu) * jax.lax.rsqrt(var + 1e-5) * g + b


def _uv(x, t):
    r = jnp.maximum(x, 0.0) + 1e-7
    v = jnp.exp(t * r)
    return v * r, v


def _prep_body(x_ref, t_ref, p_ref):
    u, v = _uv(x_ref[...], t_ref[0, 0])
    p_ref[:, :D] = u
    p_ref[:, D:] = v


def _prep(x, n, t):
    return pl.pallas_call(
        _prep_body,
        grid=(1,),
        in_specs=[
            pl.BlockSpec((n, D), lambda i: (0, 0)),
            pl.BlockSpec((1, 1), lambda i: (0, 0)),
        ],
        out_specs=pl.BlockSpec((n, 2 * D), lambda i: (0, 0)),
        out_shape=jax.ShapeDtypeStruct((n, 2 * D), jnp.float32),
    )(x, t.reshape(1, 1))


def _layer1_body(a_ref, p_ref, xd_ref, w1_ref, b1_ref, g1_ref, be1_ref,
                 w2_ref, b2_ref, ng_ref, nb_ref, t2_ref, o_ref, p2_ref):
    a = a_ref[...].astype(jnp.bfloat16)
    uv = jnp.dot(a, p_ref[...].astype(jnp.bfloat16),
                 preferred_element_type=jnp.float32)
    agg = uv[:, :D] / (uv[:, D:] + 1e-16)
    h0 = agg + xd_ref[...]
    h = jnp.dot(h0, w1_ref[...], preferred_element_type=jnp.float32) + b1_ref[...]
    h = jnp.maximum(_ln(h, g1_ref[...], be1_ref[...]), 0.0)
    y = jnp.dot(h, w2_ref[...], preferred_element_type=jnp.float32) + b2_ref[...]
    hg = jax.nn.gelu(_ln(y, ng_ref[...], nb_ref[...]))
    o_ref[...] = hg
    u2, v2 = _uv(hg, t2_ref[0, 0])
    p2_ref[:, :D] = u2
    p2_ref[:, D:] = v2


def _layer2_body(a_ref, p_ref, xd_ref, w1_ref, b1_ref, g1_ref, be1_ref,
                 w2_ref, b2_ref, o_ref):
    a = a_ref[...].astype(jnp.bfloat16)
    uv = jnp.dot(a, p_ref[...].astype(jnp.bfloat16),
                 preferred_element_type=jnp.float32)
    agg = uv[:, :D] / (uv[:, D:] + 1e-16)
    h0 = agg + xd_ref[...]
    h = jnp.dot(h0, w1_ref[...], preferred_element_type=jnp.float32) + b1_ref[...]
    h = jnp.maximum(_ln(h, g1_ref[...], be1_ref[...]), 0.0)
    y = jnp.dot(h, w2_ref[...], preferred_element_type=jnp.float32) + b2_ref[...]
    m = jnp.max(y, axis=-1, keepdims=True)
    e = jnp.exp(y - m)
    o_ref[...] = y - m - jnp.log(jnp.sum(e, axis=-1, keepdims=True))


def _full(shape):
    return pl.BlockSpec(shape, lambda i: (0, 0))


def _layer1(A, P, x, W1, b1, g1, be1, W2, b2, ng, nb, t2):
    k = A.shape[1]
    return pl.pallas_call(
        _layer1_body,
        grid=(M1 // BR,),
        in_specs=[
            pl.BlockSpec((BR, k), lambda i: (i, 0)),
            _full((k, 2 * D)),
            pl.BlockSpec((BR, D), lambda i: (i, 0)),
            _full((D, HID)), _full((1, HID)), _full((1, HID)), _full((1, HID)),
            _full((HID, D)), _full((1, D)), _full((1, D)), _full((1, D)),
            _full((1, 1)),
        ],
        out_specs=(
            pl.BlockSpec((BR, D), lambda i: (i, 0)),
            pl.BlockSpec((BR, 2 * D), lambda i: (i, 0)),
        ),
        out_shape=(
            jax.ShapeDtypeStruct((M1, D), jnp.float32),
            jax.ShapeDtypeStruct((M1, 2 * D), jnp.float32),
        ),
    )(A, P, x, W1, b1.reshape(1, -1), g1.reshape(1, -1), be1.reshape(1, -1),
      W2, b2.reshape(1, -1), ng.reshape(1, -1), nb.reshape(1, -1),
      t2.reshape(1, 1))


def _layer2(A, P, xd, W1, b1, g1, be1, W2, b2):
    k = A.shape[1]
    return pl.pallas_call(
        _layer2_body,
        grid=(M2 // BR,),
        in_specs=[
            pl.BlockSpec((BR, k), lambda i: (i, 0)),
            _full((k, 2 * D)),
            pl.BlockSpec((BR, D), lambda i: (i, 0)),
            _full((D, HID)), _full((1, HID)), _full((1, HID)), _full((1, HID)),
            _full((HID, OUT)), _full((1, OUT)),
        ],
        out_specs=pl.BlockSpec((BR, OUT), lambda i: (i, 0)),
        out_shape=jax.ShapeDtypeStruct((M2, OUT), jnp.float32),
    )(A, P, xd, W1, b1.reshape(1, -1), g1.reshape(1, -1), be1.reshape(1, -1),
      W2, b2.reshape(1, -1))


def _sc_phase_body(src_hbm, dst_hbm, out_hbm, e0, e1, idx, ones, zbuf, slab,
                   *, ec, k_src, rounds, slab_n, ch, zb):
    c = lax.axis_index("c")
    s = lax.axis_index("s")
    nchunks = ec // ch
    stripe = slab_n // 16

    def fill(buf, n, val):
        def f(i, _):
            for u in range(5):
                b = (i * 5 + u) * 16
                buf[pl.ds(b, 16)] = jnp.full((16,), val, jnp.float32)
            return 0
        lax.fori_loop(0, n // 80, f, 0)

    fill(ones, ch, 1.0)
    fill(zbuf, zb, 0.0)
    for q in range(nchunks):
        pltpu.sync_copy(src_hbm.at[pl.ds(s * ec + q * ch, ch)], e0)
        pltpu.sync_copy(dst_hbm.at[pl.ds(s * ec + q * ch, ch)],
                        e1.at[pl.ds(q * ch, ch)])

        def prep_body(i, _):
            for u in range(5):
                b = (i * 5 + u) * 16
                e1[pl.ds(q * ch + b, 16)] = (
                    e1[pl.ds(q * ch + b, 16)] * k_src + e0[pl.ds(b, 16)])
            return 0

        lax.fori_loop(0, ch // 80, prep_body, 0)

    for r in range(rounds):
        for j in range(stripe // zb):
            pltpu.sync_copy(zbuf, slab.at[pl.ds(s * stripe + j * zb, zb)])
        plsc.subcore_barrier()
        base = (c * rounds + r) * slab_n
        for q in range(nchunks):
            def idx_body(i, _):
                for u in range(5):
                    b = (i * 5 + u) * 16
                    t = e1[pl.ds(q * ch + b, 16)] - base
                    ok = (t >= 0) & (t < slab_n)
                    dmp = slab_n + (t & (DUMP_N - 1))
                    idx[pl.ds(b, 16)] = jnp.where(ok, t, dmp)
                return 0

            lax.fori_loop(0, ch // 80, idx_body, 0)
            pltpu.sync_copy(ones, slab.at[idx], add=True)
        plsc.subcore_barrier()
        pltpu.sync_copy(
            slab.at[pl.ds(s * stripe, stripe)],
            out_hbm.at[pl.ds(base + s * stripe, stripe)])
        plsc.subcore_barrier()


def _sc_counts(src_arr, dst_arr, m_dst, k_src, rounds, slab_n, ch, zb):
    ec = src_arr.shape[0] // 16
    body = functools.partial(_sc_phase_body, ec=ec, k_src=k_src,
                             rounds=rounds, slab_n=slab_n, ch=ch, zb=zb)
    mesh = plsc.VectorSubcoreMesh(core_axis_name="c", subcore_axis_name="s")
    flat = pl.kernel(
        body,
        out_type=jax.ShapeDtypeStruct((m_dst * k_src,), jnp.float32),
        mesh=mesh,
        scratch_types=[
            pltpu.VMEM((ch,), jnp.int32),
            pltpu.VMEM((ec,), jnp.int32),
            pltpu.VMEM((ch,), jnp.int32),
            pltpu.VMEM((ch,), jnp.float32),
            pltpu.VMEM((zb,), jnp.float32),
            pltpu.VMEM_SHARED((slab_n + DUMP_N,), jnp.float32),
        ],
    )(src_arr, dst_arr)
    return flat.reshape(m_dst, k_src)


def _pad_edges(ei, n_pad, dst_pad):
    p = n_pad - ei.shape[1]
    return jnp.concatenate(
        [ei, jnp.stack([jnp.arange(p, dtype=ei.dtype) & (DUMP_N - 1),
                        jnp.full((p,), dst_pad, ei.dtype)])], axis=1)


def _build_counts(edge_index1, edge_index2):
    ei1 = _pad_edges(edge_index1, E1P, N1)
    ei2 = _pad_edges(edge_index2, E2P, K2)
    a1 = _sc_counts(ei1[0], ei1[1], M1, K1, R1_, SLAB1, CH1, ZB1)
    a2 = _sc_counts(ei2[0], ei2[1], M2, K2, R2_, SLAB2, CH2, ZB2)
    return a1, a2


def kernel(x, edge_index1, edge_index2, t1, W1a, b1a, g1a, be1a, W1b, b1b,
           ng, nb, t2, W2a, b2a, g2a, be2a, W2b, b2b):
    A1, A2 = _build_counts(edge_index1, edge_index2)
    P1 = _prep(x, K1, t1)
    hg, P2 = _layer1(A1, P1, x, W1a, b1a, g1a, be1a, W1b, b1b, ng, nb, t2)
    out = _layer2(A2, P2, hg, W2a, b2a, g2a, be2a, W2b, b2b)
    return out[:N2]


# speedup vs baseline: 1.0191x; 1.0191x over previous
import functools

import jax
import jax.numpy as jnp
from jax import lax
from jax.experimental import pallas as pl
from jax.experimental.pallas import tpu as pltpu
from jax.experimental.pallas import tpu_sc as plsc

N0, N1, N2 = 10000, 5000, 2500
D, HID, OUT = 128, 256, 64
K1 = 5120
M1 = 2560
K2 = 2560
M2 = 2560
BR = 512

DUMP_N = 16384
E1P, E2P = 327680, 163840
SLAB1, R1_, CH1, ZB1 = 256 * K1, 5, 5120, 10240
SLAB2, R2_, CH2, ZB2 = 640 * K2, 2, 2560, 5120


def _ln(h, g, b):
    mu = jnp.mean(h, axis=-1, keepdims=True)
    var = jnp.mean((h - mu) * (h - mu), axis=-1, keepdims=True)
    return (h - mu) * jax.lax.rsqrt(var + 1e-5) * g + b


def _uv(x, t):
    r = jnp.maximum(x, 0.0) + 1e-7
    v = jnp.exp(t * r)
    return v * r, v


def _prep_body(x_ref, t_ref, p_ref):
    u, v = _uv(x_ref[...], t_ref[0, 0])
    p_ref[:, :D] = u
    p_ref[:, D:] = v


def _prep(x, n, t):
    return pl.pallas_call(
        _prep_body,
        grid=(1,),
        in_specs=[
            pl.BlockSpec((n, D), lambda i: (0, 0)),
            pl.BlockSpec((1, 1), lambda i: (0, 0)),
        ],
        out_specs=pl.BlockSpec((n, 2 * D), lambda i: (0, 0)),
        out_shape=jax.ShapeDtypeStruct((n, 2 * D), jnp.float32),
    )(x, t.reshape(1, 1))


def _layer1_body(a_ref, p_ref, xd_ref, w1_ref, b1_ref, g1_ref, be1_ref,
                 w2_ref, b2_ref, ng_ref, nb_ref, t2_ref, o_ref, p2_ref):
    a = a_ref[...].astype(jnp.bfloat16)
    uv = jnp.dot(a, p_ref[...].astype(jnp.bfloat16),
                 preferred_element_type=jnp.float32)
    agg = uv[:, :D] / (uv[:, D:] + 1e-16)
    h0 = agg + xd_ref[...]
    h = jnp.dot(h0, w1_ref[...], preferred_element_type=jnp.float32) + b1_ref[...]
    h = jnp.maximum(_ln(h, g1_ref[...], be1_ref[...]), 0.0)
    y = jnp.dot(h, w2_ref[...], preferred_element_type=jnp.float32) + b2_ref[...]
    hg = jax.nn.gelu(_ln(y, ng_ref[...], nb_ref[...]))
    o_ref[...] = hg
    u2, v2 = _uv(hg, t2_ref[0, 0])
    p2_ref[:, :D] = u2
    p2_ref[:, D:] = v2


def _layer2_body(a_ref, p_ref, xd_ref, w1_ref, b1_ref, g1_ref, be1_ref,
                 w2_ref, b2_ref, o_ref):
    a = a_ref[...].astype(jnp.bfloat16)
    uv = jnp.dot(a, p_ref[...].astype(jnp.bfloat16),
                 preferred_element_type=jnp.float32)
    agg = uv[:, :D] / (uv[:, D:] + 1e-16)
    h0 = agg + xd_ref[...]
    h = jnp.dot(h0, w1_ref[...], preferred_element_type=jnp.float32) + b1_ref[...]
    h = jnp.maximum(_ln(h, g1_ref[...], be1_ref[...]), 0.0)
    y = jnp.dot(h, w2_ref[...], preferred_element_type=jnp.float32) + b2_ref[...]
    m = jnp.max(y, axis=-1, keepdims=True)
    e = jnp.exp(y - m)
    o_ref[...] = y - m - jnp.log(jnp.sum(e, axis=-1, keepdims=True))


def _full(shape):
    return pl.BlockSpec(shape, lambda i: (0, 0))


def _layer1(A, P, x, W1, b1, g1, be1, W2, b2, ng, nb, t2):
    k = A.shape[1]
    return pl.pallas_call(
        _layer1_body,
        grid=(M1 // BR,),
        in_specs=[
            pl.BlockSpec((BR, k), lambda i: (i, 0)),
            _full((k, 2 * D)),
            pl.BlockSpec((BR, D), lambda i: (i, 0)),
            _full((D, HID)), _full((1, HID)), _full((1, HID)), _full((1, HID)),
            _full((HID, D)), _full((1, D)), _full((1, D)), _full((1, D)),
            _full((1, 1)),
        ],
        out_specs=(
            pl.BlockSpec((BR, D), lambda i: (i, 0)),
            pl.BlockSpec((BR, 2 * D), lambda i: (i, 0)),
        ),
        out_shape=(
            jax.ShapeDtypeStruct((M1, D), jnp.float32),
            jax.ShapeDtypeStruct((M1, 2 * D), jnp.float32),
        ),
    )(A, P, x, W1, b1.reshape(1, -1), g1.reshape(1, -1), be1.reshape(1, -1),
      W2, b2.reshape(1, -1), ng.reshape(1, -1), nb.reshape(1, -1),
      t2.reshape(1, 1))


def _layer2(A, P, xd, W1, b1, g1, be1, W2, b2):
    k = A.shape[1]
    return pl.pallas_call(
        _layer2_body,
        grid=(M2 // BR,),
        in_specs=[
            pl.BlockSpec((BR, k), lambda i: (i, 0)),
            _full((k, 2 * D)),
            pl.BlockSpec((BR, D), lambda i: (i, 0)),
            _full((D, HID)), _full((1, HID)), _full((1, HID)), _full((1, HID)),
            _full((HID, OUT)), _full((1, OUT)),
        ],
        out_specs=pl.BlockSpec((BR, OUT), lambda i: (i, 0)),
        out_shape=jax.ShapeDtypeStruct((M2, OUT), jnp.float32),
    )(A, P, xd, W1, b1.reshape(1, -1), g1.reshape(1, -1), be1.reshape(1, -1),
      W2, b2.reshape(1, -1))


def _sc_phase_body(src_hbm, dst_hbm, out_hbm, e0, e1, idx, ones, zbuf, slab,
                   *, ec, k_src, rounds, slab_n, ch, zb):
    c = lax.axis_index("c")
    s = lax.axis_index("s")
    nchunks = ec // ch
    stripe = slab_n // 16

    def fill(buf, n, val):
        def f(i, _):
            for u in range(5):
                b = (i * 5 + u) * 16
                buf[pl.ds(b, 16)] = jnp.full((16,), val, jnp.float32)
            return 0
        lax.fori_loop(0, n // 80, f, 0)

    fill(ones, ch, 1.0)
    fill(zbuf, zb, 0.0)
    for q in range(nchunks):
        pltpu.sync_copy(src_hbm.at[pl.ds(s * ec + q * ch, ch)], e0)
        pltpu.sync_copy(dst_hbm.at[pl.ds(s * ec + q * ch, ch)],
                        e1.at[pl.ds(q * ch, ch)])

        def prep_body(i, _):
            for u in range(5):
                b = (i * 5 + u) * 16
                e1[pl.ds(q * ch + b, 16)] = (
                    e1[pl.ds(q * ch + b, 16)] * k_src + e0[pl.ds(b, 16)])
            return 0

        lax.fori_loop(0, ch // 80, prep_body, 0)

    for r in range(rounds):
        for j in range(stripe // zb):
            pltpu.sync_copy(zbuf, slab.at[pl.ds(s * stripe + j * zb, zb)])
        plsc.subcore_barrier()
        base = (c * rounds + r) * slab_n
        for q in range(nchunks):
            def idx_body(i, _):
                for u in range(5):
                    b = (i * 5 + u) * 16
                    t = e1[pl.ds(q * ch + b, 16)] - base
                    ok = (t >= 0) & (t < slab_n)
                    dmp = slab_n + (t & (DUMP_N - 1))
                    idx[pl.ds(b, 16)] = jnp.where(ok, t, dmp)
                return 0

            lax.fori_loop(0, ch // 80, idx_body, 0)
            pltpu.sync_copy(ones, slab.at[idx], add=True)
        plsc.subcore_barrier()
        pltpu.sync_copy(
            slab.at[pl.ds(s * stripe, stripe)],
            out_hbm.at[pl.ds(base + s * stripe, stripe)])
        plsc.subcore_barrier()


def _sc_counts(src_arr, dst_arr, m_dst, k_src, rounds, slab_n, ch, zb):
    ec = src_arr.shape[0] // 16
    body = functools.partial(_sc_phase_body, ec=ec, k_src=k_src,
                             rounds=rounds, slab_n=slab_n, ch=ch, zb=zb)
    mesh = plsc.VectorSubcoreMesh(core_axis_name="c", subcore_axis_name="s")
    flat = pl.kernel(
        body,
        out_type=jax.ShapeDtypeStruct((m_dst * k_src,), jnp.float32),
        mesh=mesh,
        scratch_types=[
            pltpu.VMEM((ch,), jnp.int32),
            pltpu.VMEM((ec,), jnp.int32),
            pltpu.VMEM((ch,), jnp.int32),
            pltpu.VMEM((ch,), jnp.float32),
            pltpu.VMEM((zb,), jnp.float32),
            pltpu.VMEM_SHARED((slab_n + DUMP_N,), jnp.float32),
        ],
    )(src_arr, dst_arr)
    return flat.reshape(m_dst, k_src)


def _pad_edges(ei, n_pad, dst_pad):
    p = n_pad - ei.shape[1]
    return jnp.concatenate(
        [ei, jnp.stack([jnp.arange(p, dtype=ei.dtype) & (DUMP_N - 1),
                        jnp.full((p,), dst_pad, ei.dtype)])], axis=1)


def _build_counts(edge_index1, edge_index2):
    ei1 = _pad_edges(edge_index1, E1P, N1)
    ei2 = _pad_edges(edge_index2, E2P, K2)
    a1 = _sc_counts(ei1[0], ei1[1], M1, K1, R1_, SLAB1, CH1, ZB1)
    a2 = _sc_counts(ei2[0], ei2[1], M2, K2, R2_, SLAB2, CH2, ZB2)
    return a1, a2


def kernel(x, edge_index1, edge_index2, t1, W1a, b1a, g1a, be1a, W1b, b1b,
           ng, nb, t2, W2a, b2a, g2a, be2a, W2b, b2b):
    A1, A2 = _build_counts(edge_index1, edge_index2)
    P1 = _prep(x, K1, t1)
    hg, P2 = _layer1(A1, P1, x, W1a, b1a, g1a, be1a, W1b, b1b, ng, nb, t2)
    out = _layer2(A2, P2, hg, W2a, b2a, g2a, be2a, W2b, b2b)
    return out[:N2]


# speedup vs baseline: 1.0247x; 1.0054x over previous
import functools

import jax
import jax.numpy as jnp
from jax import lax
from jax.experimental import pallas as pl
from jax.experimental.pallas import tpu as pltpu
from jax.experimental.pallas import tpu_sc as plsc

N0, N1, N2 = 10000, 5000, 2500
D, HID, OUT = 128, 256, 64
K1 = 5120
M1 = 2560
K2 = 2560
M2 = 2560
BR = 640

DUMP_N = 16384
E1P, E2P = 327680, 163840
SLAB1, R1_, CH1, ZB1 = 256 * K1, 5, 5120, 10240
SLAB2, R2_, CH2, ZB2 = 640 * K2, 2, 2560, 5120


def _ln(h, g, b):
    mu = jnp.mean(h, axis=-1, keepdims=True)
    var = jnp.mean((h - mu) * (h - mu), axis=-1, keepdims=True)
    return (h - mu) * jax.lax.rsqrt(var + 1e-5) * g + b


def _uv(x, t):
    r = jnp.maximum(x, 0.0) + 1e-7
    v = jnp.exp(t * r)
    return v * r, v


def _prep_body(x_ref, t_ref, p_ref):
    u, v = _uv(x_ref[...], t_ref[0, 0])
    p_ref[:, :D] = u
    p_ref[:, D:] = v


def _prep(x, n, t):
    return pl.pallas_call(
        _prep_body,
        grid=(1,),
        in_specs=[
            pl.BlockSpec((n, D), lambda i: (0, 0)),
            pl.BlockSpec((1, 1), lambda i: (0, 0)),
        ],
        out_specs=pl.BlockSpec((n, 2 * D), lambda i: (0, 0)),
        out_shape=jax.ShapeDtypeStruct((n, 2 * D), jnp.float32),
    )(x, t.reshape(1, 1))


def _layer1_body(a_ref, p_ref, xd_ref, w1_ref, b1_ref, g1_ref, be1_ref,
                 w2_ref, b2_ref, ng_ref, nb_ref, t2_ref, o_ref, p2_ref):
    a = a_ref[...].astype(jnp.bfloat16)
    uv = jnp.dot(a, p_ref[...].astype(jnp.bfloat16),
                 preferred_element_type=jnp.float32)
    agg = uv[:, :D] / (uv[:, D:] + 1e-16)
    h0 = agg + xd_ref[...]
    h = jnp.dot(h0, w1_ref[...], preferred_element_type=jnp.float32) + b1_ref[...]
    h = jnp.maximum(_ln(h, g1_ref[...], be1_ref[...]), 0.0)
    y = jnp.dot(h, w2_ref[...], preferred_element_type=jnp.float32) + b2_ref[...]
    hg = jax.nn.gelu(_ln(y, ng_ref[...], nb_ref[...]))
    o_ref[...] = hg
    u2, v2 = _uv(hg, t2_ref[0, 0])
    p2_ref[:, :D] = u2
    p2_ref[:, D:] = v2


def _layer2_body(a_ref, p_ref, xd_ref, w1_ref, b1_ref, g1_ref, be1_ref,
                 w2_ref, b2_ref, o_ref):
    a = a_ref[...].astype(jnp.bfloat16)
    uv = jnp.dot(a, p_ref[...].astype(jnp.bfloat16),
                 preferred_element_type=jnp.float32)
    agg = uv[:, :D] / (uv[:, D:] + 1e-16)
    h0 = agg + xd_ref[...]
    h = jnp.dot(h0, w1_ref[...], preferred_element_type=jnp.float32) + b1_ref[...]
    h = jnp.maximum(_ln(h, g1_ref[...], be1_ref[...]), 0.0)
    y = jnp.dot(h, w2_ref[...], preferred_element_type=jnp.float32) + b2_ref[...]
    m = jnp.max(y, axis=-1, keepdims=True)
    e = jnp.exp(y - m)
    o_ref[...] = y - m - jnp.log(jnp.sum(e, axis=-1, keepdims=True))


def _full(shape):
    return pl.BlockSpec(shape, lambda i: (0, 0))


def _layer1(A, P, x, W1, b1, g1, be1, W2, b2, ng, nb, t2):
    k = A.shape[1]
    return pl.pallas_call(
        _layer1_body,
        grid=(M1 // BR,),
        in_specs=[
            pl.BlockSpec((BR, k), lambda i: (i, 0)),
            _full((k, 2 * D)),
            pl.BlockSpec((BR, D), lambda i: (i, 0)),
            _full((D, HID)), _full((1, HID)), _full((1, HID)), _full((1, HID)),
            _full((HID, D)), _full((1, D)), _full((1, D)), _full((1, D)),
            _full((1, 1)),
        ],
        out_specs=(
            pl.BlockSpec((BR, D), lambda i: (i, 0)),
            pl.BlockSpec((BR, 2 * D), lambda i: (i, 0)),
        ),
        out_shape=(
            jax.ShapeDtypeStruct((M1, D), jnp.float32),
            jax.ShapeDtypeStruct((M1, 2 * D), jnp.float32),
        ),
    )(A, P, x, W1, b1.reshape(1, -1), g1.reshape(1, -1), be1.reshape(1, -1),
      W2, b2.reshape(1, -1), ng.reshape(1, -1), nb.reshape(1, -1),
      t2.reshape(1, 1))


def _layer2(A, P, xd, W1, b1, g1, be1, W2, b2):
    k = A.shape[1]
    return pl.pallas_call(
        _layer2_body,
        grid=(M2 // BR,),
        in_specs=[
            pl.BlockSpec((BR, k), lambda i: (i, 0)),
            _full((k, 2 * D)),
            pl.BlockSpec((BR, D), lambda i: (i, 0)),
            _full((D, HID)), _full((1, HID)), _full((1, HID)), _full((1, HID)),
            _full((HID, OUT)), _full((1, OUT)),
        ],
        out_specs=pl.BlockSpec((BR, OUT), lambda i: (i, 0)),
        out_shape=jax.ShapeDtypeStruct((M2, OUT), jnp.float32),
    )(A, P, xd, W1, b1.reshape(1, -1), g1.reshape(1, -1), be1.reshape(1, -1),
      W2, b2.reshape(1, -1))


def _sc_phase_body(src_hbm, dst_hbm, out_hbm, e0, e1, idx, ones, zbuf, slab,
                   *, ec, k_src, rounds, slab_n, ch, zb):
    c = lax.axis_index("c")
    s = lax.axis_index("s")
    nchunks = ec // ch
    stripe = slab_n // 16

    def fill(buf, n, val):
        def f(i, _):
            for u in range(5):
                b = (i * 5 + u) * 16
                buf[pl.ds(b, 16)] = jnp.full((16,), val, jnp.float32)
            return 0
        lax.fori_loop(0, n // 80, f, 0)

    fill(ones, ch, 1.0)
    fill(zbuf, zb, 0.0)
    for q in range(nchunks):
        pltpu.sync_copy(src_hbm.at[pl.ds(s * ec + q * ch, ch)], e0)
        pltpu.sync_copy(dst_hbm.at[pl.ds(s * ec + q * ch, ch)],
                        e1.at[pl.ds(q * ch, ch)])

        def prep_body(i, _):
            for u in range(5):
                b = (i * 5 + u) * 16
                e1[pl.ds(q * ch + b, 16)] = (
                    e1[pl.ds(q * ch + b, 16)] * k_src + e0[pl.ds(b, 16)])
            return 0

        lax.fori_loop(0, ch // 80, prep_body, 0)

    for r in range(rounds):
        for j in range(stripe // zb):
            pltpu.sync_copy(zbuf, slab.at[pl.ds(s * stripe + j * zb, zb)])
        plsc.subcore_barrier()
        base = (c * rounds + r) * slab_n
        for q in range(nchunks):
            def idx_body(i, _):
                for u in range(5):
                    b = (i * 5 + u) * 16
                    t = e1[pl.ds(q * ch + b, 16)] - base
                    ok = (t >= 0) & (t < slab_n)
                    dmp = slab_n + (t & (DUMP_N - 1))
                    idx[pl.ds(b, 16)] = jnp.where(ok, t, dmp)
                return 0

            lax.fori_loop(0, ch // 80, idx_body, 0)
            pltpu.sync_copy(ones, slab.at[idx], add=True)
        plsc.subcore_barrier()
        pltpu.sync_copy(
            slab.at[pl.ds(s * stripe, stripe)],
            out_hbm.at[pl.ds(base + s * stripe, stripe)])
        plsc.subcore_barrier()


def _sc_counts(src_arr, dst_arr, m_dst, k_src, rounds, slab_n, ch, zb):
    ec = src_arr.shape[0] // 16
    body = functools.partial(_sc_phase_body, ec=ec, k_src=k_src,
                             rounds=rounds, slab_n=slab_n, ch=ch, zb=zb)
    mesh = plsc.VectorSubcoreMesh(core_axis_name="c", subcore_axis_name="s")
    flat = pl.kernel(
        body,
        out_type=jax.ShapeDtypeStruct((m_dst * k_src,), jnp.float32),
        mesh=mesh,
        scratch_types=[
            pltpu.VMEM((ch,), jnp.int32),
            pltpu.VMEM((ec,), jnp.int32),
            pltpu.VMEM((ch,), jnp.int32),
            pltpu.VMEM((ch,), jnp.float32),
            pltpu.VMEM((zb,), jnp.float32),
            pltpu.VMEM_SHARED((slab_n + DUMP_N,), jnp.float32),
        ],
    )(src_arr, dst_arr)
    return flat.reshape(m_dst, k_src)


def _pad_edges(ei, n_pad, dst_pad):
    p = n_pad - ei.shape[1]
    return jnp.concatenate(
        [ei, jnp.stack([jnp.arange(p, dtype=ei.dtype) & (DUMP_N - 1),
                        jnp.full((p,), dst_pad, ei.dtype)])], axis=1)


def _build_counts(edge_index1, edge_index2):
    ei1 = _pad_edges(edge_index1, E1P, N1)
    ei2 = _pad_edges(edge_index2, E2P, K2)
    a1 = _sc_counts(ei1[0], ei1[1], M1, K1, R1_, SLAB1, CH1, ZB1)
    a2 = _sc_counts(ei2[0], ei2[1], M2, K2, R2_, SLAB2, CH2, ZB2)
    return a1, a2


def kernel(x, edge_index1, edge_index2, t1, W1a, b1a, g1a, be1a, W1b, b1b,
           ng, nb, t2, W2a, b2a, g2a, be2a, W2b, b2b):
    A1, A2 = _build_counts(edge_index1, edge_index2)
    P1 = _prep(x, K1, t1)
    hg, P2 = _layer1(A1, P1, x, W1a, b1a, g1a, be1a, W1b, b1b, ng, nb, t2)
    out = _layer2(A2, P2, hg, W2a, b2a, g2a, be2a, W2b, b2b)
    return out[:N2]
